# baseline (speedup 1.0000x reference)
import functools

import jax
import jax.numpy as jnp
from jax import lax
from jax.experimental import pallas as pl
from jax.experimental.pallas import tpu as pltpu
from jax.experimental.pallas import tpu_sc as plsc

NC = 2
NS = 16
CHUNK = 128
HALF = 5000
HPAD = 5120
DUMP = HPAD - 1


def _sc_mesh():
    return plsc.VectorSubcoreMesh(core_axis_name="c", subcore_axis_name="s")


def _make_deg_kernel(npad, n_chunk_rows, rpt):
    cpt = n_chunk_rows // NC // NS
    assert rpt % 16 == 0

    @functools.partial(
        pl.kernel, mesh=_sc_mesh(),
        out_type=jax.ShapeDtypeStruct((NC, npad), jnp.float32),
        compiler_params=pltpu.CompilerParams(needs_layout_passes=False),
        scratch_types=[
            pltpu.VMEM((cpt, CHUNK), jnp.int32),
            pltpu.VMEM((npad,), jnp.float32),
            pltpu.VMEM((rpt,), jnp.float32),
            pltpu.VMEM((rpt,), jnp.float32),
            pltpu.VMEM_SHARED((NS, npad), jnp.float32),
        ],
    )
    def deg_kernel(col_hbm, zn_hbm, out_hbm, idx2d, cnt, tmp, accv, shared):
        cid = lax.axis_index("c")
        sid = lax.axis_index("s")
        cb = cid * (n_chunk_rows // NC) + sid * cpt
        pltpu.sync_copy(col_hbm.at[pl.ds(cb, cpt)], idx2d)
        pltpu.sync_copy(zn_hbm, cnt)
        ones16 = jnp.ones((16,), jnp.float32)

        def step(r, carry):
            for c in range(CHUNK // 16):
                idx = idx2d[r, pl.ds(c * 16, 16)]
                plsc.addupdate_scatter(cnt, [idx], ones16)
            return carry

        lax.fori_loop(0, cpt, step, 0)
        pltpu.sync_copy(cnt, shared.at[sid])
        plsc.subcore_barrier()
        pltpu.sync_copy(shared.at[0, pl.ds(sid * rpt, rpt)], accv)

        def red(t, carry):
            pltpu.sync_copy(shared.at[t, pl.ds(sid * rpt, rpt)], tmp)

            def vadd(v, c2):
                sl = pl.ds(v * 16, 16)
                accv[sl] = accv[sl] + tmp[sl]
                return c2

            lax.fori_loop(0, rpt // 16, vadd, 0)
            return carry

        lax.fori_loop(1, NS, red, 0)
        pltpu.sync_copy(accv, out_hbm.at[cid, pl.ds(sid * rpt, rpt)])

    return deg_kernel


def _make_agg_kernel(cr_core, rpt):
    cpt = cr_core // NS
    NB = 4
    assert cpt % NB == 0

    @functools.partial(
        pl.kernel, mesh=_sc_mesh(),
        out_type=jax.ShapeDtypeStruct((NC, 2, HPAD, 128), jnp.float32),
        scratch_types=[
            pltpu.VMEM((cpt, CHUNK), jnp.int32),
            pltpu.VMEM((cpt, CHUNK), jnp.int32),
            [pltpu.VMEM((CHUNK, 128), jnp.float32)] * NB,
            pltpu.VMEM_SHARED((HPAD, 128), jnp.float32),
            [pltpu.SemaphoreType.DMA] * NB,
            [pltpu.SemaphoreType.DMA] * NB,
        ],
    )
    def agg_kernel(table_hbm, row_hbm, col_hbm, z_hbm, out_hbm,
                   row_v, col_v, g, acc, gs, ss):
        cid = lax.axis_index("c")
        sid = lax.axis_index("s")
        cb = sid * cpt
        pltpu.sync_copy(row_hbm.at[cid, pl.ds(cb, cpt)], row_v)

        for p in range(2):
            pltpu.sync_copy(col_hbm.at[p, cid, pl.ds(cb, cpt)], col_v)
            pltpu.sync_copy(z_hbm.at[pl.ds(sid * rpt, rpt)],
                            acc.at[pl.ds(sid * rpt, rpt)])
            plsc.subcore_barrier()

            for b in range(NB):
                pltpu.async_copy(table_hbm.at[row_v.at[b]], g[b], gs[b])

            def step(j, carry):
                base = NB * j
                for b in range(NB):
                    pltpu.make_async_copy(
                        table_hbm.at[row_v.at[base + b]], g[b], gs[b]).wait()
                    pltpu.async_copy(
                        g[b], acc.at[col_v.at[base + b]], ss[b], add=True)

                @pl.when(j < cpt // NB - 1)
                def _():
                    for b in range(NB):
                        pltpu.make_async_copy(
                            g[b], acc.at[col_v.at[base + b]], ss[b]).wait()
                        pltpu.async_copy(
                            table_hbm.at[row_v.at[base + NB + b]], g[b], gs[b])

                return carry

            lax.fori_loop(0, cpt // NB, step, 0)
            for b in range(NB):
                pltpu.make_async_copy(
                    g[b], acc.at[col_v.at[cpt - NB + b]], ss[b]).wait()
            plsc.subcore_barrier()
            pltpu.sync_copy(acc.at[pl.ds(sid * rpt, rpt)],
                            out_hbm.at[cid, p, pl.ds(sid * rpt, rpt)])

    return agg_kernel


def kernel(x, edge_index, W1, b1, W2, b2):
    n, f_in = x.shape
    h_dim = W1.shape[1]
    c_dim = W2.shape[1]
    e = edge_index.shape[1]
    fh = h_dim // 2
    assert f_in % 128 == 0 and fh == 128 and n == 2 * HALF

    npad = (n + 1 + 255) // 256 * 256
    rpt_deg = npad // NS
    rpt = HPAD // NS
    epad = (e + CHUNK * 32 - 1) // (CHUNK * 32) * (CHUNK * 32)
    ncr = epad // CHUNK
    bn = 1000
    n_row_blocks = n // bn
    pb = HALF // bn

    ei = edge_index.astype(jnp.int32)
    pad_e = epad - e
    row_p = jnp.concatenate([ei[0], jnp.zeros((pad_e,), jnp.int32)])
    col_p = jnp.concatenate([ei[1], jnp.full((pad_e,), n, jnp.int32)])
    col2d = col_p.reshape(ncr, CHUNK)
    cm = [jnp.where((col_p >= p * HALF) & (col_p < (p + 1) * HALF),
                    col_p - p * HALF, DUMP).reshape(ncr, CHUNK)
          for p in range(2)]
    row_l1 = jnp.stack([row_p, row_p + n]).reshape(NC, ncr, CHUNK)
    col_l1 = jnp.broadcast_to(jnp.stack(cm)[:, None], (2, NC, ncr, CHUNK))
    row_l2 = row_p.reshape(NC, ncr // 2, CHUNK)
    col_l2 = jnp.stack(cm).reshape(2, NC, ncr // 2, CHUNK)
    zn = jnp.zeros((npad,), jnp.float32)
    zf = jnp.zeros((HPAD, 128), jnp.float32)
    b1r = b1.reshape(1, h_dim)
    b2r = b2.reshape(1, c_dim)

    deg_parts = _make_deg_kernel(npad, ncr, rpt_deg)(col2d, zn)
    deg_cnt = deg_parts[:, :n, None]

    def xw1_body(x_ref, w_ref, dp_ref, xws_ref, d_ref):
        cnt = dp_ref[0] + dp_ref[1]
        dd = lax.rsqrt(cnt + 2.0)
        xw = jnp.dot(x_ref[...], w_ref[...], preferred_element_type=jnp.float32)
        xws_ref[0] = xw * dd
        d_ref[...] = dd

    xws1, d_vec = pl.pallas_call(
        xw1_body,
        grid=(2, n_row_blocks),
        in_specs=[
            pl.BlockSpec((bn, f_in), lambda j, i: (i, 0)),
            pl.BlockSpec((f_in, fh), lambda j, i: (0, j)),
            pl.BlockSpec((NC, bn, 1), lambda j, i: (0, i, 0)),
        ],
        out_specs=[
            pl.BlockSpec((1, bn, fh), lambda j, i: (j, i, 0)),
            pl.BlockSpec((bn, 1), lambda j, i: (i, 0)),
        ],
        out_shape=[
            jax.ShapeDtypeStruct((2, n, fh), jnp.float32),
            jax.ShapeDtypeStruct((n, 1), jnp.float32),
        ],
    )(x, W1, deg_cnt)

    xws1_flat = xws1.reshape(2 * n, fh)
    agg1 = _make_agg_kernel(ncr, rpt)(xws1_flat, row_l1, col_l1, zf)

    def layer2_body(agg_ref, xws_ref, d_ref, b1_ref, w2_ref, out_ref):
        dd = d_ref[...]
        h0 = jnp.maximum(
            dd * (agg_ref[0, 0] + 2.0 * xws_ref[0]) + b1_ref[0:1, :fh], 0.0)
        h1 = jnp.maximum(
            dd * (agg_ref[1, 0] + 2.0 * xws_ref[1]) + b1_ref[0:1, fh:], 0.0)
        xw2 = (jnp.dot(h0, w2_ref[:fh, :], preferred_element_type=jnp.float32)
               + jnp.dot(h1, w2_ref[fh:, :], preferred_element_type=jnp.float32))
        out_ref[...] = jnp.concatenate(
            [dd * xw2, jnp.zeros((bn, 128 - c_dim), jnp.float32)], axis=1)

    xws2 = pl.pallas_call(
        layer2_body,
        grid=(n_row_blocks,),
        in_specs=[
            pl.BlockSpec((NC, 1, bn, fh), lambda i: (0, i // pb, i % pb, 0)),
            pl.BlockSpec((2, bn, fh), lambda i: (0, i, 0)),
            pl.BlockSpec((bn, 1), lambda i: (i, 0)),
            pl.BlockSpec((1, h_dim), lambda i: (0, 0)),
            pl.BlockSpec((h_dim, c_dim), lambda i: (0, 0)),
        ],
        out_specs=pl.BlockSpec((bn, 128), lambda i: (i, 0)),
        out_shape=jax.ShapeDtypeStruct((n, 128), jnp.float32),
    )(agg1, xws1, d_vec, b1r, W2)

    agg2 = _make_agg_kernel(ncr // 2, rpt)(xws2, row_l2, col_l2, zf)

    def out_body(agg_ref, xws_ref, d_ref, b2_ref, out_ref):
        dd = d_ref[...]
        z = dd * (agg_ref[0, 0][:, :c_dim] + agg_ref[1, 0][:, :c_dim]
                  + 2.0 * xws_ref[:, :c_dim]) + b2_ref[...]
        m = jnp.max(z, axis=1, keepdims=True)
        lse = jnp.log(jnp.sum(jnp.exp(z - m), axis=1, keepdims=True))
        out_ref[...] = z - m - lse

    out = pl.pallas_call(
        out_body,
        grid=(n_row_blocks,),
        in_specs=[
            pl.BlockSpec((NC, 1, bn, 128), lambda i: (0, i // pb, i % pb, 0)),
            pl.BlockSpec((bn, 128), lambda i: (i, 0)),
            pl.BlockSpec((bn, 1), lambda i: (i, 0)),
            pl.BlockSpec((1, c_dim), lambda i: (0, 0)),
        ],
        out_specs=pl.BlockSpec((bn, c_dim), lambda i: (i, 0)),
        out_shape=jax.ShapeDtypeStruct((n, c_dim), jnp.float32),
    )(agg2, xws2, d_vec, b2r)

    return out


# speedup vs baseline: 1.0383x; 1.0383x over previous
import functools

import jax
import jax.numpy as jnp
from jax import lax
from jax.experimental import pallas as pl
from jax.experimental.pallas import tpu as pltpu
from jax.experimental.pallas import tpu_sc as plsc

NC = 2
NS = 16
CHUNK = 128
HALF = 5000
HPAD = 5120
DUMP = HPAD - 1


def _sc_mesh():
    return plsc.VectorSubcoreMesh(core_axis_name="c", subcore_axis_name="s")


def _make_deg_kernel(npad, n_chunk_rows, rpt):
    cpt = n_chunk_rows // NC // NS
    assert rpt % 16 == 0

    @functools.partial(
        pl.kernel, mesh=_sc_mesh(),
        out_type=jax.ShapeDtypeStruct((NC, npad), jnp.float32),
        compiler_params=pltpu.CompilerParams(needs_layout_passes=False),
        scratch_types=[
            pltpu.VMEM((cpt, CHUNK), jnp.int32),
            pltpu.VMEM((npad,), jnp.float32),
            pltpu.VMEM((rpt,), jnp.float32),
            pltpu.VMEM((rpt,), jnp.float32),
            pltpu.VMEM_SHARED((NS, npad), jnp.float32),
        ],
    )
    def deg_kernel(col_hbm, zn_hbm, out_hbm, idx2d, cnt, tmp, accv, shared):
        cid = lax.axis_index("c")
        sid = lax.axis_index("s")
        cb = cid * (n_chunk_rows // NC) + sid * cpt
        pltpu.sync_copy(col_hbm.at[pl.ds(cb, cpt)], idx2d)
        pltpu.sync_copy(zn_hbm, cnt)
        ones16 = jnp.ones((16,), jnp.float32)

        def step(r, carry):
            for c in range(CHUNK // 16):
                idx = idx2d[r, pl.ds(c * 16, 16)]
                plsc.addupdate_scatter(cnt, [idx], ones16)
            return carry

        lax.fori_loop(0, cpt, step, 0)
        pltpu.sync_copy(cnt, shared.at[sid])
        plsc.subcore_barrier()
        pltpu.sync_copy(shared.at[0, pl.ds(sid * rpt, rpt)], accv)

        def red(t, carry):
            pltpu.sync_copy(shared.at[t, pl.ds(sid * rpt, rpt)], tmp)

            def vadd(v, c2):
                sl = pl.ds(v * 16, 16)
                accv[sl] = accv[sl] + tmp[sl]
                return c2

            lax.fori_loop(0, rpt // 16, vadd, 0)
            return carry

        lax.fori_loop(1, NS, red, 0)
        pltpu.sync_copy(accv, out_hbm.at[cid, pl.ds(sid * rpt, rpt)])

    return deg_kernel


def _make_agg_kernel(cr_core, rpt):
    cpt = cr_core // NS
    assert cpt % 2 == 0

    @functools.partial(
        pl.kernel, mesh=_sc_mesh(),
        out_type=jax.ShapeDtypeStruct((NC, 2, HPAD, 128), jnp.float32),
        scratch_types=[
            pltpu.VMEM((cpt, CHUNK), jnp.int32),
            pltpu.VMEM((cpt, CHUNK), jnp.int32),
            pltpu.VMEM((CHUNK, 128), jnp.float32),
            pltpu.VMEM((CHUNK, 128), jnp.float32),
            pltpu.VMEM_SHARED((HPAD, 128), jnp.float32),
            pltpu.SemaphoreType.DMA,
            pltpu.SemaphoreType.DMA,
        ],
    )
    def agg_kernel(table_hbm, row_hbm, col_hbm, z_hbm, out_hbm,
                   row_v, col_v, g0, g1, acc, sem0, sem1):
        cid = lax.axis_index("c")
        sid = lax.axis_index("s")
        cb = sid * cpt
        pltpu.sync_copy(row_hbm.at[cid, pl.ds(cb, cpt)], row_v)

        for p in range(2):
            pltpu.sync_copy(col_hbm.at[p, cid, pl.ds(cb, cpt)], col_v)
            pltpu.sync_copy(z_hbm.at[pl.ds(sid * rpt, rpt)],
                            acc.at[pl.ds(sid * rpt, rpt)])
            plsc.subcore_barrier()

            pltpu.async_copy(table_hbm.at[row_v.at[0]], g0, sem0)

            def step(j, carry):
                i0 = 2 * j
                i1 = 2 * j + 1
                pltpu.async_copy(table_hbm.at[row_v.at[i1]], g1, sem1)
                pltpu.make_async_copy(
                    table_hbm.at[row_v.at[i0]], g0, sem0).wait()
                pltpu.sync_copy(g0, acc.at[col_v.at[i0]], add=True)

                @pl.when(j < cpt // 2 - 1)
                def _():
                    pltpu.async_copy(table_hbm.at[row_v.at[i0 + 2]], g0, sem0)

                pltpu.make_async_copy(
                    table_hbm.at[row_v.at[i1]], g1, sem1).wait()
                pltpu.sync_copy(g1, acc.at[col_v.at[i1]], add=True)
                return carry

            lax.fori_loop(0, cpt // 2, step, 0)
            plsc.subcore_barrier()
            pltpu.sync_copy(acc.at[pl.ds(sid * rpt, rpt)],
                            out_hbm.at[cid, p, pl.ds(sid * rpt, rpt)])

    return agg_kernel


def kernel(x, edge_index, W1, b1, W2, b2):
    n, f_in = x.shape
    h_dim = W1.shape[1]
    c_dim = W2.shape[1]
    e = edge_index.shape[1]
    fh = h_dim // 2
    assert f_in % 128 == 0 and fh == 128 and n == 2 * HALF

    npad = (n + 1 + 255) // 256 * 256
    rpt_deg = npad // NS
    rpt = HPAD // NS
    epad = (e + CHUNK * 32 - 1) // (CHUNK * 32) * (CHUNK * 32)
    ncr = epad // CHUNK
    bn = 1000
    n_row_blocks = n // bn
    pb = HALF // bn

    ei = edge_index.astype(jnp.int32)
    pad_e = epad - e
    row_p = jnp.concatenate([ei[0], jnp.zeros((pad_e,), jnp.int32)])
    col_p = jnp.concatenate([ei[1], jnp.full((pad_e,), n, jnp.int32)])
    col2d = col_p.reshape(ncr, CHUNK)
    dumps = HALF + jnp.arange(epad, dtype=jnp.int32) % (HPAD - HALF)
    cm = [jnp.where((col_p >= p * HALF) & (col_p < (p + 1) * HALF),
                    col_p - p * HALF, dumps).reshape(ncr, CHUNK)
          for p in range(2)]
    row_l1 = jnp.stack([row_p, row_p + n]).reshape(NC, ncr, CHUNK)
    col_l1 = jnp.broadcast_to(jnp.stack(cm)[:, None], (2, NC, ncr, CHUNK))
    row_l2 = row_p.reshape(NC, ncr // 2, CHUNK)
    col_l2 = jnp.stack(cm).reshape(2, NC, ncr // 2, CHUNK)
    zn = jnp.zeros((npad,), jnp.float32)
    zf = jnp.zeros((HPAD, 128), jnp.float32)
    b1r = b1.reshape(1, h_dim)
    b2r = b2.reshape(1, c_dim)

    deg_parts = _make_deg_kernel(npad, ncr, rpt_deg)(col2d, zn)
    deg_cnt = deg_parts[:, :n, None]

    def xw1_body(x_ref, w_ref, dp_ref, xws_ref, d_ref):
        cnt = dp_ref[0] + dp_ref[1]
        dd = lax.rsqrt(cnt + 2.0)
        xw = jnp.dot(x_ref[...], w_ref[...], preferred_element_type=jnp.float32)
        xws_ref[0] = xw * dd
        d_ref[...] = dd

    xws1, d_vec = pl.pallas_call(
        xw1_body,
        grid=(2, n_row_blocks),
        in_specs=[
            pl.BlockSpec((bn, f_in), lambda j, i: (i, 0)),
            pl.BlockSpec((f_in, fh), lambda j, i: (0, j)),
            pl.BlockSpec((NC, bn, 1), lambda j, i: (0, i, 0)),
        ],
        out_specs=[
            pl.BlockSpec((1, bn, fh), lambda j, i: (j, i, 0)),
            pl.BlockSpec((bn, 1), lambda j, i: (i, 0)),
        ],
        out_shape=[
            jax.ShapeDtypeStruct((2, n, fh), jnp.float32),
            jax.ShapeDtypeStruct((n, 1), jnp.float32),
        ],
    )(x, W1, deg_cnt)

    xws1_flat = xws1.reshape(2 * n, fh)
    agg1 = _make_agg_kernel(ncr, rpt)(xws1_flat, row_l1, col_l1, zf)

    def layer2_body(agg_ref, xws_ref, d_ref, b1_ref, w2_ref, out_ref):
        dd = d_ref[...]
        h0 = jnp.maximum(
            dd * (agg_ref[0, 0] + 2.0 * xws_ref[0]) + b1_ref[0:1, :fh], 0.0)
        h1 = jnp.maximum(
            dd * (agg_ref[1, 0] + 2.0 * xws_ref[1]) + b1_ref[0:1, fh:], 0.0)
        xw2 = (jnp.dot(h0, w2_ref[:fh, :], preferred_element_type=jnp.float32)
               + jnp.dot(h1, w2_ref[fh:, :], preferred_element_type=jnp.float32))
        out_ref[...] = jnp.concatenate(
            [dd * xw2, jnp.zeros((bn, 128 - c_dim), jnp.float32)], axis=1)

    xws2 = pl.pallas_call(
        layer2_body,
        grid=(n_row_blocks,),
        in_specs=[
            pl.BlockSpec((NC, 1, bn, fh), lambda i: (0, i // pb, i % pb, 0)),
            pl.BlockSpec((2, bn, fh), lambda i: (0, i, 0)),
            pl.BlockSpec((bn, 1), lambda i: (i, 0)),
            pl.BlockSpec((1, h_dim), lambda i: (0, 0)),
            pl.BlockSpec((h_dim, c_dim), lambda i: (0, 0)),
        ],
        out_specs=pl.BlockSpec((bn, 128), lambda i: (i, 0)),
        out_shape=jax.ShapeDtypeStruct((n, 128), jnp.float32),
    )(agg1, xws1, d_vec, b1r, W2)

    agg2 = _make_agg_kernel(ncr // 2, rpt)(xws2, row_l2, col_l2, zf)

    def out_body(agg_ref, xws_ref, d_ref, b2_ref, out_ref):
        dd = d_ref[...]
        z = dd * (agg_ref[0, 0][:, :c_dim] + agg_ref[1, 0][:, :c_dim]
                  + 2.0 * xws_ref[:, :c_dim]) + b2_ref[...]
        m = jnp.max(z, axis=1, keepdims=True)
        lse = jnp.log(jnp.sum(jnp.exp(z - m), axis=1, keepdims=True))
        out_ref[...] = z - m - lse

    out = pl.pallas_call(
        out_body,
        grid=(n_row_blocks,),
        in_specs=[
            pl.BlockSpec((NC, 1, bn, 128), lambda i: (0, i // pb, i % pb, 0)),
            pl.BlockSpec((bn, 128), lambda i: (i, 0)),
            pl.BlockSpec((bn, 1), lambda i: (i, 0)),
            pl.BlockSpec((1, c_dim), lambda i: (0, 0)),
        ],
        out_specs=pl.BlockSpec((bn, c_dim), lambda i: (i, 0)),
        out_shape=jax.ShapeDtypeStruct((n, c_dim), jnp.float32),
    )(agg2, xws2, d_vec, b2r)

    return out


# speedup vs baseline: 1.2885x; 1.2410x over previous
import functools

import jax
import jax.numpy as jnp
from jax import lax
from jax.experimental import pallas as pl
from jax.experimental.pallas import tpu as pltpu
from jax.experimental.pallas import tpu_sc as plsc

NC = 2
NS = 16
CHUNK = 128
HALF = 5000
HPAD = 5120
DUMP = HPAD - 1


def _sc_mesh():
    return plsc.VectorSubcoreMesh(core_axis_name="c", subcore_axis_name="s")


def _make_deg_kernel(npad, n_chunk_rows, rpt):
    cpt = n_chunk_rows // NC // NS
    assert rpt % 16 == 0

    @functools.partial(
        pl.kernel, mesh=_sc_mesh(),
        out_type=jax.ShapeDtypeStruct((NC, npad), jnp.float32),
        compiler_params=pltpu.CompilerParams(needs_layout_passes=False),
        scratch_types=[
            pltpu.VMEM((cpt, CHUNK), jnp.int32),
            pltpu.VMEM((npad,), jnp.float32),
            pltpu.VMEM((rpt,), jnp.float32),
            pltpu.VMEM((rpt,), jnp.float32),
            pltpu.VMEM_SHARED((NS, npad), jnp.float32),
        ],
    )
    def deg_kernel(col_hbm, zn_hbm, out_hbm, idx2d, cnt, tmp, accv, shared):
        cid = lax.axis_index("c")
        sid = lax.axis_index("s")
        cb = cid * (n_chunk_rows // NC) + sid * cpt
        pltpu.sync_copy(col_hbm.at[pl.ds(cb, cpt)], idx2d)
        pltpu.sync_copy(zn_hbm, cnt)
        ones16 = jnp.ones((16,), jnp.float32)

        def step(r, carry):
            for c in range(CHUNK // 16):
                idx = idx2d[r, pl.ds(c * 16, 16)]
                plsc.addupdate_scatter(cnt, [idx], ones16)
            return carry

        lax.fori_loop(0, cpt, step, 0)
        pltpu.sync_copy(cnt, shared.at[sid])
        plsc.subcore_barrier()
        pltpu.sync_copy(shared.at[0, pl.ds(sid * rpt, rpt)], accv)

        def red(t, carry):
            pltpu.sync_copy(shared.at[t, pl.ds(sid * rpt, rpt)], tmp)

            def vadd(v, c2):
                sl = pl.ds(v * 16, 16)
                accv[sl] = accv[sl] + tmp[sl]
                return c2

            lax.fori_loop(0, rpt // 16, vadd, 0)
            return carry

        lax.fori_loop(1, NS, red, 0)
        pltpu.sync_copy(accv, out_hbm.at[cid, pl.ds(sid * rpt, rpt)])

    return deg_kernel


def _make_agg_kernel(cr_core, rpt, npass=2):
    cpt = cr_core // NS
    assert cpt % 2 == 0

    @functools.partial(
        pl.kernel, mesh=_sc_mesh(),
        out_type=jax.ShapeDtypeStruct((NC, npass, HPAD, 128), jnp.float32),
        scratch_types=[
            pltpu.VMEM((cpt, CHUNK), jnp.int32),
            pltpu.VMEM((cpt, CHUNK), jnp.int32),
            pltpu.VMEM((CHUNK, 128), jnp.float32),
            pltpu.VMEM((CHUNK, 128), jnp.float32),
            pltpu.VMEM_SHARED((HPAD, 128), jnp.float32),
            pltpu.SemaphoreType.DMA,
            pltpu.SemaphoreType.DMA,
        ],
    )
    def agg_kernel(table_hbm, row_hbm, col_hbm, z_hbm, out_hbm,
                   row_v, col_v, g0, g1, acc, sem0, sem1):
        cid = lax.axis_index("c")
        sid = lax.axis_index("s")
        cb = sid * cpt
        pltpu.sync_copy(row_hbm.at[cid, pl.ds(cb, cpt)], row_v)

        for p in range(npass):
            pltpu.sync_copy(col_hbm.at[p, cid, pl.ds(cb, cpt)], col_v)
            pltpu.sync_copy(z_hbm.at[pl.ds(sid * rpt, rpt)],
                            acc.at[pl.ds(sid * rpt, rpt)])
            plsc.subcore_barrier()

            pltpu.async_copy(table_hbm.at[row_v.at[0]], g0, sem0)

            def step(j, carry):
                i0 = 2 * j
                i1 = 2 * j + 1
                pltpu.async_copy(table_hbm.at[row_v.at[i1]], g1, sem1)
                pltpu.make_async_copy(
                    table_hbm.at[row_v.at[i0]], g0, sem0).wait()
                pltpu.sync_copy(g0, acc.at[col_v.at[i0]], add=True)

                @pl.when(j < cpt // 2 - 1)
                def _():
                    pltpu.async_copy(table_hbm.at[row_v.at[i0 + 2]], g0, sem0)

                pltpu.make_async_copy(
                    table_hbm.at[row_v.at[i1]], g1, sem1).wait()
                pltpu.sync_copy(g1, acc.at[col_v.at[i1]], add=True)
                return carry

            lax.fori_loop(0, cpt // 2, step, 0)
            plsc.subcore_barrier()
            pltpu.sync_copy(acc.at[pl.ds(sid * rpt, rpt)],
                            out_hbm.at[cid, p, pl.ds(sid * rpt, rpt)])

    return agg_kernel


def kernel(x, edge_index, W1, b1, W2, b2):
    n, f_in = x.shape
    h_dim = W1.shape[1]
    c_dim = W2.shape[1]
    e = edge_index.shape[1]
    fh = h_dim // 2
    assert f_in % 128 == 0 and fh == 128 and n == 2 * HALF

    npad = (n + 1 + 255) // 256 * 256
    rpt_deg = npad // NS
    rpt = HPAD // NS
    epad = (e + CHUNK * 32 - 1) // (CHUNK * 32) * (CHUNK * 32)
    ncr = epad // CHUNK
    bn = 1000
    n_row_blocks = n // bn
    pb = HALF // bn

    ei = edge_index.astype(jnp.int32)
    pad_e = epad - e
    row_p = jnp.concatenate([ei[0], jnp.zeros((pad_e,), jnp.int32)])
    col_p = jnp.concatenate([ei[1], jnp.full((pad_e,), n, jnp.int32)])
    col2d = col_p.reshape(ncr, CHUNK)
    dumps = HALF + jnp.arange(epad, dtype=jnp.int32) % (HPAD - HALF)
    cm = [jnp.where((col_p >= p * HALF) & (col_p < (p + 1) * HALF),
                    col_p - p * HALF, dumps).reshape(ncr, CHUNK)
          for p in range(2)]
    row_l1 = jnp.stack([row_p, row_p + n]).reshape(NC, ncr, CHUNK)
    col_l1 = jnp.broadcast_to(jnp.stack(cm)[:, None], (2, NC, ncr, CHUNK))
    v_raw = col_p // HALF
    v2 = jnp.minimum(v_raw, 1)
    col2 = jnp.where(v_raw >= 2, DUMP, col_p - v2 * HALF)
    row_l2 = (row_p + v2 * n).reshape(NC, ncr // 2, CHUNK)
    col_l2 = col2.reshape(1, NC, ncr // 2, CHUNK)
    zn = jnp.zeros((npad,), jnp.float32)
    zf = jnp.zeros((HPAD, 128), jnp.float32)
    b1r = b1.reshape(1, h_dim)
    b2r = b2.reshape(1, c_dim)

    deg_parts = _make_deg_kernel(npad, ncr, rpt_deg)(col2d, zn)
    deg_cnt = deg_parts[:, :n, None]

    def xw1_body(x_ref, w_ref, dp_ref, xws_ref, d_ref):
        cnt = dp_ref[0] + dp_ref[1]
        dd = lax.rsqrt(cnt + 2.0)
        xw = jnp.dot(x_ref[...], w_ref[...], preferred_element_type=jnp.float32)
        xws_ref[0] = xw * dd
        d_ref[...] = dd

    xws1, d_vec = pl.pallas_call(
        xw1_body,
        grid=(2, n_row_blocks),
        in_specs=[
            pl.BlockSpec((bn, f_in), lambda j, i: (i, 0)),
            pl.BlockSpec((f_in, fh), lambda j, i: (0, j)),
            pl.BlockSpec((NC, bn, 1), lambda j, i: (0, i, 0)),
        ],
        out_specs=[
            pl.BlockSpec((1, bn, fh), lambda j, i: (j, i, 0)),
            pl.BlockSpec((bn, 1), lambda j, i: (i, 0)),
        ],
        out_shape=[
            jax.ShapeDtypeStruct((2, n, fh), jnp.float32),
            jax.ShapeDtypeStruct((n, 1), jnp.float32),
        ],
    )(x, W1, deg_cnt)

    xws1_flat = xws1.reshape(2 * n, fh)
    agg1 = _make_agg_kernel(ncr, rpt)(xws1_flat, row_l1, col_l1, zf)

    def layer2_body(agg_ref, xws_ref, d_ref, b1_ref, w2_ref, out_ref):
        dd = d_ref[...]
        h0 = jnp.maximum(
            dd * (agg_ref[0, 0] + 2.0 * xws_ref[0]) + b1_ref[0:1, :fh], 0.0)
        h1 = jnp.maximum(
            dd * (agg_ref[1, 0] + 2.0 * xws_ref[1]) + b1_ref[0:1, fh:], 0.0)
        xw2 = (jnp.dot(h0, w2_ref[:fh, :], preferred_element_type=jnp.float32)
               + jnp.dot(h1, w2_ref[fh:, :], preferred_element_type=jnp.float32))
        v = dd * xw2
        zc = jnp.zeros((bn, 64 - c_dim), jnp.float32)
        z64 = jnp.zeros((bn, 64), jnp.float32)
        out_ref[0] = jnp.concatenate([v, zc, z64], axis=1)
        out_ref[1] = jnp.concatenate([z64, v, zc], axis=1)

    xws2 = pl.pallas_call(
        layer2_body,
        grid=(n_row_blocks,),
        in_specs=[
            pl.BlockSpec((NC, 1, bn, fh), lambda i: (0, i // pb, i % pb, 0)),
            pl.BlockSpec((2, bn, fh), lambda i: (0, i, 0)),
            pl.BlockSpec((bn, 1), lambda i: (i, 0)),
            pl.BlockSpec((1, h_dim), lambda i: (0, 0)),
            pl.BlockSpec((h_dim, c_dim), lambda i: (0, 0)),
        ],
        out_specs=pl.BlockSpec((2, bn, 128), lambda i: (0, i, 0)),
        out_shape=jax.ShapeDtypeStruct((2, n, 128), jnp.float32),
    )(agg1, xws1, d_vec, b1r, W2)

    xws2_flat = xws2.reshape(2 * n, 128)
    agg2 = _make_agg_kernel(ncr // 2, rpt, npass=1)(
        xws2_flat, row_l2, col_l2, zf)

    def out_body(agg_ref, xws_ref, d_ref, b2_ref, out_ref):
        dd = d_ref[...]
        hi = pl.program_id(0) >= pb
        a = agg_ref[0, 0] + agg_ref[1, 0]
        ag = jnp.where(hi, a[:, 64:64 + c_dim], a[:, :c_dim])
        xw = jnp.where(hi, xws_ref[0][:, 64:64 + c_dim], xws_ref[0][:, :c_dim])
        z = dd * (ag + 2.0 * xw) + b2_ref[...]
        m = jnp.max(z, axis=1, keepdims=True)
        lse = jnp.log(jnp.sum(jnp.exp(z - m), axis=1, keepdims=True))
        out_ref[...] = z - m - lse

    out = pl.pallas_call(
        out_body,
        grid=(n_row_blocks,),
        in_specs=[
            pl.BlockSpec((NC, 1, bn, 128), lambda i: (0, 0, i % pb, 0)),
            pl.BlockSpec((1, bn, 128), lambda i: (i // pb, i, 0)),
            pl.BlockSpec((bn, 1), lambda i: (i, 0)),
            pl.BlockSpec((1, c_dim), lambda i: (0, 0)),
        ],
        out_specs=pl.BlockSpec((bn, c_dim), lambda i: (i, 0)),
        out_shape=jax.ShapeDtypeStruct((n, c_dim), jnp.float32),
    )(agg2, xws2, d_vec, b2r)

    return out
